# baseline (speedup 1.0000x reference)
import functools

import jax
import jax.numpy as jnp
from jax.experimental import pallas as pl
from jax.experimental.pallas import tpu as pltpu

TOPK = 10
STATIC_E = 2
NEG = float(-3e38)
BIGCOL = float(2**30)


def _topk_body(q_ref, k_ref, out_s_ref, out_i_ref, t_ref, ti_ref, sbuf_ref,
               *, nk, kb, n_keys):
    ki = pl.program_id(0)
    rows = q_ref.shape[0]
    col = jax.lax.broadcasted_iota(jnp.int32, (rows, kb), 1).astype(jnp.float32)
    iota_t = jax.lax.broadcasted_iota(jnp.int32, (1, TOPK), 1)

    @pl.when(ki == 0)
    def _init():
        t_ref[...] = jnp.full_like(t_ref, NEG)
        ti_ref[...] = jnp.zeros_like(ti_ref)

    s = sbuf_ref[...]

    @pl.when(ki < nk)
    def _compute():
        s_new = jax.lax.dot_general(
            q_ref[...], k_ref[...],
            (((1,), (1,)), ((), ())),
            preferred_element_type=jnp.float32,
        )
        sbuf_ref[...] = jnp.where(ki * kb + col < n_keys, s_new, NEG)

    @pl.when(ki > 0)
    def _merge():
        base = (ki - 1) * kb

        def step(carry):
            m, am, t, ti = carry
            gm = base + am.astype(jnp.int32)
            p = jnp.sum(((t > m) | ((t == m) & (ti < gm))).astype(jnp.int32),
                        axis=1, keepdims=True)
            t_sh = jnp.concatenate([t[:, :1], t[:, :-1]], axis=1)
            ti_sh = jnp.concatenate([ti[:, :1], ti[:, :-1]], axis=1)
            t = jnp.where(iota_t < p, t, jnp.where(iota_t == p, m, t_sh))
            ti = jnp.where(iota_t < p, ti, jnp.where(iota_t == p, gm, ti_sh))
            live = (s < m) | ((s == m) & (col > am))
            sm = jnp.where(live, s, NEG)
            m2 = jnp.max(sm, axis=1, keepdims=True)
            am2 = jnp.min(jnp.where(sm == m2, col, BIGCOL), axis=1,
                          keepdims=True)
            return m2, am2, t, ti

        def cond(carry):
            m, _, t, _ = carry
            return jnp.any(m > t[:, TOPK - 1:TOPK])

        m0 = jnp.max(s, axis=1, keepdims=True)
        am0 = jnp.min(jnp.where(s == m0, col, BIGCOL), axis=1, keepdims=True)
        carry = (m0, am0, t_ref[...], ti_ref[...])
        for _ in range(STATIC_E):
            carry = step(carry)
        m, am, t, ti = jax.lax.while_loop(cond, step, carry)
        t_ref[...] = t
        ti_ref[...] = ti

    @pl.when(ki == nk)
    def _emit():
        out_s_ref[...] = t_ref[...]
        out_i_ref[...] = ti_ref[...]


def kernel(queries, keys):
    n_q, dim = queries.shape
    n_keys, _ = keys.shape
    kb = min(512, n_keys)
    nk = pl.cdiv(n_keys, kb)

    body = functools.partial(_topk_body, nk=nk, kb=kb, n_keys=n_keys)
    out_s, out_i = pl.pallas_call(
        body,
        grid=(nk + 1,),
        in_specs=[
            pl.BlockSpec((n_q, dim), lambda ki: (0, 0)),
            pl.BlockSpec((kb, dim), lambda ki: (jnp.minimum(ki, nk - 1), 0)),
        ],
        out_specs=[
            pl.BlockSpec((n_q, TOPK), lambda ki: (0, 0)),
            pl.BlockSpec((n_q, TOPK), lambda ki: (0, 0)),
        ],
        out_shape=[
            jax.ShapeDtypeStruct((n_q, TOPK), jnp.float32),
            jax.ShapeDtypeStruct((n_q, TOPK), jnp.int32),
        ],
        scratch_shapes=[
            pltpu.VMEM((n_q, TOPK), jnp.float32),
            pltpu.VMEM((n_q, TOPK), jnp.int32),
            pltpu.VMEM((n_q, kb), jnp.float32),
        ],
        compiler_params=pltpu.CompilerParams(
            dimension_semantics=("arbitrary",),
        ),
    )(queries, keys)
    return out_s, out_i


# speedup vs baseline: 1.1627x; 1.1627x over previous
import functools

import jax
import jax.numpy as jnp
from jax.experimental import pallas as pl
from jax.experimental.pallas import tpu as pltpu

TOPK = 10
STATIC_E = 2
NEG = float(-3e38)
BIGCOL = float(2**30)


def _topk_body(q_ref, k_ref, out_s_ref, out_i_ref, t_ref, ti_ref, sbuf_ref,
               *, nk, kb, n_keys):
    ki = pl.program_id(0)
    rows = q_ref.shape[0]
    col = jax.lax.broadcasted_iota(jnp.int32, (rows, kb), 1).astype(jnp.float32)
    iota_t = jax.lax.broadcasted_iota(jnp.int32, (1, TOPK), 1)

    @pl.when(ki == 0)
    def _init():
        t_ref[...] = jnp.full_like(t_ref, NEG)
        ti_ref[...] = jnp.zeros_like(ti_ref)

    s = sbuf_ref[...]

    @pl.when(ki < nk)
    def _compute():
        s_new = jax.lax.dot_general(
            q_ref[...], k_ref[...],
            (((1,), (1,)), ((), ())),
            preferred_element_type=jnp.float32,
        )
        sbuf_ref[...] = jnp.where(ki * kb + col < n_keys, s_new, NEG)

    @pl.when(ki > 0)
    def _merge():
        base = (ki - 1) * kb

        def step(carry):
            m, am, t, ti = carry
            gm = base + am.astype(jnp.int32)
            p = jnp.sum(((t > m) | ((t == m) & (ti < gm))).astype(jnp.int32),
                        axis=1, keepdims=True)
            t_sh = jnp.concatenate([t[:, :1], t[:, :-1]], axis=1)
            ti_sh = jnp.concatenate([ti[:, :1], ti[:, :-1]], axis=1)
            t = jnp.where(iota_t < p, t, jnp.where(iota_t == p, m, t_sh))
            ti = jnp.where(iota_t < p, ti, jnp.where(iota_t == p, gm, ti_sh))
            live = (s < m) | ((s == m) & (col > am))
            sm = jnp.where(live, s, NEG)
            m2 = jnp.max(sm, axis=1, keepdims=True)
            am2 = jnp.min(jnp.where(sm == m2, col, BIGCOL), axis=1,
                          keepdims=True)
            return m2, am2, t, ti

        def cond(carry):
            m, _, t, _ = carry
            return jnp.any(m > t[:, TOPK - 1:TOPK])

        m0 = jnp.max(s, axis=1, keepdims=True)
        am0 = jnp.min(jnp.where(s == m0, col, BIGCOL), axis=1, keepdims=True)
        carry = (m0, am0, t_ref[...], ti_ref[...])
        for _ in range(STATIC_E):
            carry = step(carry)
        m, am, t, ti = jax.lax.while_loop(cond, step, carry)
        t_ref[...] = t
        ti_ref[...] = ti

    @pl.when(ki == nk)
    def _emit():
        out_s_ref[...] = t_ref[...]
        out_i_ref[...] = ti_ref[...]


def kernel(queries, keys):
    n_q, dim = queries.shape
    n_keys, _ = keys.shape
    kb = min(1024, n_keys)
    nk = pl.cdiv(n_keys, kb)

    body = functools.partial(_topk_body, nk=nk, kb=kb, n_keys=n_keys)
    out_s, out_i = pl.pallas_call(
        body,
        grid=(nk + 1,),
        in_specs=[
            pl.BlockSpec((n_q, dim), lambda ki: (0, 0)),
            pl.BlockSpec((kb, dim), lambda ki: (jnp.minimum(ki, nk - 1), 0)),
        ],
        out_specs=[
            pl.BlockSpec((n_q, TOPK), lambda ki: (0, 0)),
            pl.BlockSpec((n_q, TOPK), lambda ki: (0, 0)),
        ],
        out_shape=[
            jax.ShapeDtypeStruct((n_q, TOPK), jnp.float32),
            jax.ShapeDtypeStruct((n_q, TOPK), jnp.int32),
        ],
        scratch_shapes=[
            pltpu.VMEM((n_q, TOPK), jnp.float32),
            pltpu.VMEM((n_q, TOPK), jnp.int32),
            pltpu.VMEM((n_q, kb), jnp.float32),
        ],
        compiler_params=pltpu.CompilerParams(
            dimension_semantics=("arbitrary",),
        ),
    )(queries, keys)
    return out_s, out_i
